# Initial kernel scaffold; baseline (speedup 1.0000x reference)
import jax
import jax.numpy as jnp
from jax.experimental import pallas as pl


def kernel(x, embedding_input, g1_w, g1_b, g2_w, g2_b):
    raise NotImplementedError("write your pallas kernel here")


# speedup vs baseline: 14.0307x; 14.0307x over previous
import functools

import numpy as np
import jax
import jax.numpy as jnp
from jax.experimental import pallas as pl
from jax.experimental.pallas import tpu as pltpu

_B = 10
_H = 400
_MAIN_IN = 128
_MAIN_OUT = 64
_SEG_EA = [_MAIN_IN * _H, _H, _H * _H, _H, _H * _H, _H, _H * _MAIN_OUT, _MAIN_OUT]
_SEG_START = [0, 51200, 51600, 211600, 212000, 372000, 372400, 398000]
_SEG_SHAPES = [
    (_B, _H, _MAIN_IN), (_B, _H), (_B, _H, _H), (_B, _H),
    (_B, _H, _H), (_B, _H), (_B, _MAIN_OUT, _H), (_B, _MAIN_OUT),
]
_TOTAL = sum(_SEG_EA)
_NSEG = len(_SEG_EA)

_W = 512
_NSTEPS = -(-_TOTAL // _W)


def _gumbel_noise() -> jax.Array:
    base = jax.random.key(42)
    parts = [
        jax.random.gumbel(jax.random.fold_in(base, i), (_B, ea), jnp.float32)
        for i, ea in enumerate(_SEG_EA)
    ]
    return jnp.concatenate(parts, axis=1)


def _mm_body(emb_ref, g1w_ref, g1b_ref, g2w_ref, g2b_ref, g_ref, z_ref):
    h = jnp.maximum(
        jax.lax.dot_general(emb_ref[...], g1w_ref[...], (((1,), (1,)), ((), ())),
                            preferred_element_type=jnp.float32) + g1b_ref[...],
        0.0)
    z_ref[...] = (
        jax.lax.dot_general(h, g2w_ref[...], (((1,), (1,)), ((), ())),
                            preferred_element_type=jnp.float32)
        + g2b_ref[...] + g_ref[...])


def _mm_call(emb, g1w, g1b, g2w, g2b, gnoise):
    return pl.pallas_call(
        _mm_body,
        grid=(_NSTEPS,),
        in_specs=[
            pl.BlockSpec((_B, 10), lambda i: (0, 0)),
            pl.BlockSpec((256, 10), lambda i: (0, 0)),
            pl.BlockSpec((1, 256), lambda i: (0, 0)),
            pl.BlockSpec((_W, 256), lambda i: (i, 0)),
            pl.BlockSpec((1, _W), lambda i: (0, i)),
            pl.BlockSpec((_B, _W), lambda i: (0, i)),
        ],
        out_specs=pl.BlockSpec((_B, _W), lambda i: (0, i)),
        out_shape=jax.ShapeDtypeStruct((_B, _TOTAL), jnp.float32),
    )(emb, g1w, g1b, g2w, g2b, gnoise)


def _sel_body(z_ref, *refs):
    out_refs = refs[:_NSEG]
    s_ref = refs[_NSEG]
    for j, ea in enumerate(_SEG_EA):
        k = ea // 2
        si = _SEG_START[j]
        b = jax.lax.bitcast_convert_type(z_ref[:, si:si + ea], jnp.int32)
        s = jnp.where(b >= 0, b, b ^ jnp.int32(0x7FFFFFFF))
        s_ref[:, :ea] = s
        cnt0 = jnp.sum((s >= 0).astype(jnp.int32), axis=1, keepdims=True)
        t0 = jnp.where(cnt0 >= k, jnp.int32(0), jnp.int32(-2147483648))

        def body(t, thr, ea=ea, k=k, s_ref=s_ref):
            cand = thr | (jnp.int32(1) << (30 - t))
            cnt = jnp.sum((s_ref[:, :ea] >= cand).astype(jnp.int32),
                          axis=1, keepdims=True)
            return jnp.where(cnt >= k, cand, thr)

        thr = jax.lax.fori_loop(0, 31, body, t0)
        out_refs[j][...] = (s_ref[:, :ea] >= thr).astype(jnp.float32)


def _sel_call(z):
    return pl.pallas_call(
        _sel_body,
        out_shape=[jax.ShapeDtypeStruct((_B, ea), jnp.float32) for ea in _SEG_EA],
        scratch_shapes=[pltpu.VMEM((_B, max(_SEG_EA)), jnp.int32)],
        compiler_params=pltpu.CompilerParams(vmem_limit_bytes=100 * 1024 * 1024),
    )(z)


def kernel(x, embedding_input, g1_w, g1_b, g2_w, g2_b):
    del x
    z = _mm_call(embedding_input, g1_w, g1_b.reshape(1, 256), g2_w,
                 g2_b.reshape(1, _TOTAL), _gumbel_noise())
    ms = _sel_call(z)
    return tuple(m.reshape(shp) for m, shp in zip(ms, _SEG_SHAPES))


# speedup vs baseline: 24.2944x; 1.7315x over previous
import functools

import numpy as np
import jax
import jax.numpy as jnp
from jax.experimental import pallas as pl
from jax.experimental.pallas import tpu as pltpu

_B = 10
_H = 400
_MAIN_IN = 128
_MAIN_OUT = 64
_SEG_EA = [_MAIN_IN * _H, _H, _H * _H, _H, _H * _H, _H, _H * _MAIN_OUT, _MAIN_OUT]
_SEG_START = [0, 51200, 51600, 211600, 212000, 372000, 372400, 398000]
_SEG_SHAPES = [
    (_B, _H, _MAIN_IN), (_B, _H), (_B, _H, _H), (_B, _H),
    (_B, _H, _H), (_B, _H), (_B, _MAIN_OUT, _H), (_B, _MAIN_OUT),
]
_TOTAL = sum(_SEG_EA)
_NSEG = len(_SEG_EA)

_W = 8192
_NSTEPS = -(-_TOTAL // _W)


def _gumbel_noise() -> jax.Array:
    base = jax.random.key(42)
    parts = [
        jax.random.gumbel(jax.random.fold_in(base, i), (_B, ea), jnp.float32)
        for i, ea in enumerate(_SEG_EA)
    ]
    return jnp.concatenate(parts, axis=1)


def _mm_body(emb_ref, g1w_ref, g1b_ref, g2w_ref, g2b_ref, g_ref, z_ref):
    h = jnp.maximum(
        jax.lax.dot_general(emb_ref[...], g1w_ref[...], (((1,), (1,)), ((), ())),
                            preferred_element_type=jnp.float32) + g1b_ref[...],
        0.0)
    z_ref[...] = (
        jax.lax.dot_general(h, g2w_ref[...], (((1,), (1,)), ((), ())),
                            preferred_element_type=jnp.float32)
        + g2b_ref[...] + g_ref[...])


def _mm_call(emb, g1w, g1b, g2w, g2b, gnoise):
    return pl.pallas_call(
        _mm_body,
        grid=(_NSTEPS,),
        in_specs=[
            pl.BlockSpec((_B, 10), lambda i: (0, 0)),
            pl.BlockSpec((256, 10), lambda i: (0, 0)),
            pl.BlockSpec((1, 256), lambda i: (0, 0)),
            pl.BlockSpec((_W, 256), lambda i: (i, 0)),
            pl.BlockSpec((1, _W), lambda i: (0, i)),
            pl.BlockSpec((_B, _W), lambda i: (0, i)),
        ],
        out_specs=pl.BlockSpec((_B, _W), lambda i: (0, i)),
        out_shape=jax.ShapeDtypeStruct((_B, _TOTAL), jnp.float32),
    )(emb, g1w, g1b, g2w, g2b, gnoise)


def _sel_body(z_ref, *refs):
    out_refs = refs[:_NSEG]
    s_ref = refs[_NSEG]
    for j, ea in enumerate(_SEG_EA):
        k = ea // 2
        si = _SEG_START[j]
        b = jax.lax.bitcast_convert_type(z_ref[:, si:si + ea], jnp.int32)
        s = jnp.where(b >= 0, b, b ^ jnp.int32(0x7FFFFFFF))
        s_ref[:, :ea] = s
        cnt0 = jnp.sum((s >= 0).astype(jnp.int32), axis=1, keepdims=True)
        t0 = jnp.where(cnt0 >= k, jnp.int32(0), jnp.int32(-2147483648))

        def body(t, thr, ea=ea, k=k, s_ref=s_ref):
            cand = thr | (jnp.int32(1) << (30 - t))
            cnt = jnp.sum((s_ref[:, :ea] >= cand).astype(jnp.int32),
                          axis=1, keepdims=True)
            return jnp.where(cnt >= k, cand, thr)

        thr = jax.lax.fori_loop(0, 31, body, t0)
        out_refs[j][...] = (s_ref[:, :ea] >= thr).astype(jnp.float32)


def _sel_call(z):
    return pl.pallas_call(
        _sel_body,
        out_shape=[jax.ShapeDtypeStruct((_B, ea), jnp.float32) for ea in _SEG_EA],
        scratch_shapes=[pltpu.VMEM((_B, max(_SEG_EA)), jnp.int32)],
        compiler_params=pltpu.CompilerParams(vmem_limit_bytes=100 * 1024 * 1024),
    )(z)


def kernel(x, embedding_input, g1_w, g1_b, g2_w, g2_b):
    del x
    z = _mm_call(embedding_input, g1_w, g1_b.reshape(1, 256), g2_w,
                 g2_b.reshape(1, _TOTAL), _gumbel_noise())
    ms = _sel_call(z)
    return tuple(m.reshape(shp) for m, shp in zip(ms, _SEG_SHAPES))
